# baseline (speedup 1.0000x reference)
import dataclasses
import functools

import jax
import jax.numpy as jnp
from jax import lax
from jax.experimental import pallas as pl
from jax.experimental.pallas import tpu as pltpu
from jax.experimental.pallas import tpu_sc as plsc

N = 10000
E = 320000
D = 128
XP = 16
NC = 2
NS = 16
NW = NC * NS
EP = E // NW
C = 80
NCHUNK = EP // C
NP = 10240
NSTRIPE = NP // NS

def _sc_params():
    cp = pltpu.CompilerParams()
    if "needs_layout_passes" in pltpu.CompilerParams.__dataclass_fields__:
        cp = dataclasses.replace(cp, needs_layout_passes=False)
    return cp


def _dg(v, idx16):
    return lax.gather(
        v, idx16[:, None],
        lax.GatherDimensionNumbers(offset_dims=(), collapsed_slice_dims=(0,),
                                   start_index_map=(0,)),
        (1,), mode=lax.GatherScatterMode.PROMISE_IN_BOUNDS)


_mesh_cache = []


def _vector_mesh():
    if not _mesh_cache:
        _mesh_cache.append(
            plsc.VectorSubcoreMesh(core_axis_name="c", subcore_axis_name="s"))
    return _mesh_cache[0]


def _pq_body(h_ref, wab_ref, p_ref, q_ref):
    pq = jnp.dot(h_ref[...], wab_ref[...], preferred_element_type=jnp.float32)
    p_ref[...] = pq[:, :D]
    q_ref[...] = pq[:, D:]


def _pq(h, wab):
    bn = 2000
    grid = (N // bn,)
    return pl.pallas_call(
        _pq_body,
        grid=grid,
        in_specs=[
            pl.BlockSpec((bn, D), lambda i: (i, 0)),
            pl.BlockSpec((D, 2 * D), lambda i: (0, 0)),
        ],
        out_specs=[
            pl.BlockSpec((bn, D), lambda i: (i, 0)),
            pl.BlockSpec((bn, D), lambda i: (i, 0)),
        ],
        out_shape=[
            jax.ShapeDtypeStruct((N, D), jnp.float32),
            jax.ShapeDtypeStruct((N, D), jnp.float32),
        ],
    )(h, wab)


def _gather_body(p_hbm, q_hbm, x4_hbm, row_hbm, col_hbm, seq_hbm,
                 g_hbm, dx_hbm,
                 idr, idc, sqg, pbuf, qbuf, x4v, dxbuf, g_sh, sem):
    cid = lax.axis_index("c")
    sid = lax.axis_index("s")
    base = (sid * NC + cid) * EP
    sbase = sid * C
    pltpu.sync_copy(seq_hbm.at[pl.ds(sbase, C)], sqg)
    pltpu.sync_copy(x4_hbm, x4v)
    iota16 = lax.iota(jnp.int32, 16)
    lane = iota16 & 3
    pats = [u * 4 + (iota16 >> 2) for u in range(4)]

    @pl.loop(0, NCHUNK)
    def _(k):
        off = base + k * C
        pltpu.sync_copy(row_hbm.at[pl.ds(off, C)], idr)
        pltpu.sync_copy(col_hbm.at[pl.ds(off, C)], idc)
        cp1 = pltpu.async_copy(p_hbm.at[idr], pbuf, sem)
        cp2 = pltpu.async_copy(q_hbm.at[idc], qbuf, sem)

        @pl.loop(0, C // 16)
        def _(g):
            r16 = idr[pl.ds(g * 16, 16)]
            c16 = idc[pl.ds(g * 16, 16)]
            for u in range(4):
                pos_r = _dg(r16, pats[u]) * 4 + lane
                pos_c = _dg(c16, pats[u]) * 4 + lane
                xr = plsc.load_gather(x4v, [pos_r])
                xc = plsc.load_gather(x4v, [pos_c])
                dxbuf[pl.ds((g * 4 + u) * 16, 16)] = xr - xc

        pltpu.sync_copy(dxbuf, dx_hbm.at[pl.ds(off * 4, C * 4)])
        cp1.wait()
        pltpu.sync_copy(pbuf, g_sh.at[pl.ds(sbase, C)])
        cp2.wait()
        pltpu.sync_copy(qbuf, g_sh.at[sqg], add=True)
        pltpu.sync_copy(g_sh.at[pl.ds(sbase, C)], g_hbm.at[pl.ds(off, C)])


def _gather(p, q, x4flat, row, col, seq):
    f = pl.kernel(
        _gather_body,
        out_type=[
            jax.ShapeDtypeStruct((E, D), jnp.float32),
            jax.ShapeDtypeStruct((E * 4,), jnp.float32),
        ],
        mesh=_vector_mesh(),
        compiler_params=_sc_params(),
        scratch_types=[
            pltpu.VMEM((C,), jnp.int32),
            pltpu.VMEM((C,), jnp.int32),
            pltpu.VMEM((C,), jnp.int32),
            pltpu.VMEM((C, D), jnp.float32),
            pltpu.VMEM((C, D), jnp.float32),
            pltpu.VMEM((N * 4,), jnp.float32),
            pltpu.VMEM((C * 4,), jnp.float32),
            pltpu.VMEM_SHARED((NS * C, D), jnp.float32),
            pltpu.SemaphoreType.DMA,
        ],
    )
    return f(p, q, x4flat, row, col, seq)


def _edge_body(g_ref, dx_ref, ea_ref, wea_ref, wd_ref, be1_ref,
               we2_ref, be2_ref, wc1_ref, bc1_ref, wc2_ref, bc2_ref,
               m_ref, wdx_ref):
    dx = dx_ref[...]
    dist = jnp.sqrt(jnp.sum(dx * dx, axis=1, keepdims=True))
    z1 = (g_ref[...]
          + jnp.dot(ea_ref[...], wea_ref[...], preferred_element_type=jnp.float32)
          + dist * wd_ref[...]
          + be1_ref[...])
    a1 = z1 * jax.nn.sigmoid(z1)
    m = jnp.dot(a1.astype(jnp.bfloat16), we2_ref[...].astype(jnp.bfloat16),
                preferred_element_type=jnp.float32) + be2_ref[...]
    m_ref[...] = m
    z2 = jnp.dot(m.astype(jnp.bfloat16), wc1_ref[...].astype(jnp.bfloat16),
                 preferred_element_type=jnp.float32) + bc1_ref[...]
    t = z2 * jax.nn.sigmoid(z2)
    cw = jnp.sum(t * wc2_ref[...], axis=1, keepdims=True) + bc2_ref[...]
    wdx_ref[...] = dx * cw


def _edge(g, dx, ea, wea, wd, be1, we2, be2, wc1, bc1, wc2, bc2):
    be = 2000
    grid = (E // be,)
    full = lambda i: (0, 0)
    return pl.pallas_call(
        _edge_body,
        grid=grid,
        in_specs=[
            pl.BlockSpec((be, D), lambda i: (i, 0)),
            pl.BlockSpec((be, 4), lambda i: (i, 0)),
            pl.BlockSpec((be, 16), lambda i: (i, 0)),
            pl.BlockSpec((16, D), full),
            pl.BlockSpec((1, D), full),
            pl.BlockSpec((1, D), full),
            pl.BlockSpec((D, D), full),
            pl.BlockSpec((1, D), full),
            pl.BlockSpec((D, D), full),
            pl.BlockSpec((1, D), full),
            pl.BlockSpec((1, D), full),
            pl.BlockSpec((1, 1), full),
        ],
        out_specs=[
            pl.BlockSpec((be, D), lambda i: (i, 0)),
            pl.BlockSpec((be, 4), lambda i: (i, 0)),
        ],
        out_shape=[
            jax.ShapeDtypeStruct((E, D), jnp.float32),
            jax.ShapeDtypeStruct((E, 4), jnp.float32),
        ],
    )(g, dx, ea, wea, wd, be1, we2, be2, wc1, bc1, wc2, bc2)


def _scatter_body(m_hbm, row_hbm, z128_hbm,
                  mp_hbm,
                  m_sh, idx, mbuf):
    cid = lax.axis_index("c")
    sid = lax.axis_index("s")
    base = (sid * NC + cid) * EP
    stripe = sid * NSTRIPE
    pltpu.sync_copy(z128_hbm, m_sh.at[pl.ds(stripe, NSTRIPE)])
    plsc.subcore_barrier()

    @pl.loop(0, NCHUNK)
    def _(k):
        off = base + k * C
        pltpu.sync_copy(row_hbm.at[pl.ds(off, C)], idx)
        pltpu.sync_copy(m_hbm.at[pl.ds(off, C)], mbuf)
        pltpu.sync_copy(mbuf, m_sh.at[idx], add=True)

    plsc.subcore_barrier()
    pltpu.sync_copy(m_sh.at[pl.ds(stripe, NSTRIPE)],
                    mp_hbm.at[cid, pl.ds(stripe, NSTRIPE)])


def _scatter(m_ij, row, z128):
    f = pl.kernel(
        _scatter_body,
        out_type=jax.ShapeDtypeStruct((NC, NP, D), jnp.float32),
        mesh=_vector_mesh(),
        scratch_types=[
            pltpu.VMEM_SHARED((NP, D), jnp.float32),
            pltpu.VMEM((C,), jnp.int32),
            pltpu.VMEM((C, D), jnp.float32),
        ],
    )
    return f(m_ij, row, z128)


def _cscatter_body(w4_hbm, row_hbm, cp_hbm, idx, wvm, cacc):
    cid = lax.axis_index("c")
    sid = lax.axis_index("s")
    wid = sid * NC + cid
    base = wid * EP
    zero16 = jnp.zeros((16,), jnp.float32)

    @pl.loop(0, (N * 4) // 16)
    def _(i):
        cacc[pl.ds(i * 16, 16)] = zero16

    iota16 = lax.iota(jnp.int32, 16)
    lane = iota16 & 3
    mask4 = iota16 < 4
    vpats = [u * 4 + lane for u in range(4)]

    @pl.loop(0, NCHUNK)
    def _(k):
        off = base + k * C
        pltpu.sync_copy(row_hbm.at[pl.ds(off, C)], idx)
        pltpu.sync_copy(w4_hbm.at[pl.ds(off * 4, C * 4)], wvm)

        @pl.loop(0, C // 16)
        def _(g):
            rows16 = idx[pl.ds(g * 16, 16)]
            for t in range(4):
                vload = wvm[pl.ds((g * 4 + t) * 16, 16)]
                for u in range(4):
                    j = t * 4 + u
                    r = _dg(rows16, jnp.full((16,), j, jnp.int32))
                    v = _dg(vload, vpats[u])
                    plsc.addupdate_scatter(cacc, [r * 4 + lane], v, mask=mask4)

    pltpu.sync_copy(cacc, cp_hbm.at[wid])


def _cscatter(w4flat, row):
    f = pl.kernel(
        _cscatter_body,
        out_type=jax.ShapeDtypeStruct((NW, N * 4), jnp.float32),
        mesh=_vector_mesh(),
        compiler_params=_sc_params(),
        scratch_types=[
            pltpu.VMEM((C,), jnp.int32),
            pltpu.VMEM((C * 4,), jnp.float32),
            pltpu.VMEM((N * 4,), jnp.float32),
        ],
    )
    return f(w4flat, row)


def _node_body(h_ref, m0_ref, m1_ref, a_ref, b_ref, bn1_ref, wn2_ref, bn2_ref,
               hn_ref):
    h = h_ref[...]
    mi = m0_ref[...] + m1_ref[...]
    z = (jnp.dot(h, a_ref[...], preferred_element_type=jnp.float32)
         + jnp.dot(mi, b_ref[...], preferred_element_type=jnp.float32)
         + bn1_ref[...])
    u = z * jax.nn.sigmoid(z)
    hn_ref[...] = h + jnp.dot(u, wn2_ref[...], preferred_element_type=jnp.float32) + bn2_ref[...]


def _node(h, m0, m1, a, b, bn1, wn2, bn2):
    bn = 2000
    grid = (N // bn,)
    full = lambda i: (0, 0)
    return pl.pallas_call(
        _node_body,
        grid=grid,
        in_specs=[
            pl.BlockSpec((bn, D), lambda i: (i, 0)),
            pl.BlockSpec((bn, D), lambda i: (i, 0)),
            pl.BlockSpec((bn, D), lambda i: (i, 0)),
            pl.BlockSpec((D, D), full),
            pl.BlockSpec((D, D), full),
            pl.BlockSpec((1, D), full),
            pl.BlockSpec((D, D), full),
            pl.BlockSpec((1, D), full),
        ],
        out_specs=pl.BlockSpec((bn, D), lambda i: (i, 0)),
        out_shape=jax.ShapeDtypeStruct((N, D), jnp.float32),
    )(h, m0, m1, a, b, bn1, wn2, bn2)


def _coord_body(x4_ref, cp_ref, xn_ref):
    xn_ref[...] = x4_ref[...] + jnp.sum(cp_ref[...], axis=0)


def _coord(x4flat, cp):
    return pl.pallas_call(
        _coord_body,
        grid=(1,),
        in_specs=[
            pl.BlockSpec((N * 4,), lambda i: (0,)),
            pl.BlockSpec((NW, N * 4), lambda i: (0, 0)),
        ],
        out_specs=pl.BlockSpec((N * 4,), lambda i: (0,)),
        out_shape=jax.ShapeDtypeStruct((N * 4,), jnp.float32),
    )(x4flat, cp)


def kernel(h, x, edge_index, edge_attr,
           W_e1, b_e1, W_e2, b_e2,
           W_n1, b_n1, W_n2, b_n2,
           W_c1, b_c1, W_c2, b_c2):
    row = edge_index[0]
    col = edge_index[1]
    x4flat = jnp.pad(x, ((0, 0), (0, 1))).reshape(N * 4)
    wab = jnp.concatenate([W_e1[:D], W_e1[D:2 * D]], axis=1)
    wea = W_e1[2 * D:2 * D + 16]
    wd = W_e1[2 * D + 16:].reshape(1, D)
    seq = jnp.arange(NS * C, dtype=jnp.int32)
    z128 = jnp.zeros((NSTRIPE, D), jnp.float32)

    p, q = _pq(h, wab)
    g, dxflat = _gather(p, q, x4flat, row, col, seq)
    dx = dxflat.reshape(E, 4)
    m_ij, wdx = _edge(
        g, dx, edge_attr, wea, wd, b_e1.reshape(1, D),
        W_e2, b_e2.reshape(1, D), W_c1, b_c1.reshape(1, D),
        W_c2.reshape(1, D), b_c2.reshape(1, 1))
    mp = _scatter(m_ij, row, z128)
    cp = _cscatter(wdx.reshape(E * 4), row)
    hn = _node(
        h, mp[0, :N], mp[1, :N], W_n1[:D], W_n1[D:], b_n1.reshape(1, D),
        W_n2, b_n2.reshape(1, D))
    xn = _coord(x4flat, cp)
    return (hn, xn.reshape(N, 4)[:, :3])


# speedup vs baseline: 1.0610x; 1.0610x over previous
import dataclasses
import functools

import jax
import jax.numpy as jnp
from jax import lax
from jax.experimental import pallas as pl
from jax.experimental.pallas import tpu as pltpu
from jax.experimental.pallas import tpu_sc as plsc

N = 10000
E = 320000
D = 128
NC = 2
NS = 16
NW = NC * NS
NP = 10240
NSTRIPE = NP // NS
ES1 = 192000
ES2 = E - ES1
CS1 = 48
CS2 = 32


def _sc_params():
    cp = pltpu.CompilerParams()
    if "needs_layout_passes" in pltpu.CompilerParams.__dataclass_fields__:
        cp = dataclasses.replace(cp, needs_layout_passes=False)
    return cp


def _dg(v, idx16):
    return lax.gather(
        v, idx16[:, None],
        lax.GatherDimensionNumbers(offset_dims=(), collapsed_slice_dims=(0,),
                                   start_index_map=(0,)),
        (1,), mode=lax.GatherScatterMode.PROMISE_IN_BOUNDS)


_mesh_cache = []


def _vector_mesh():
    if not _mesh_cache:
        _mesh_cache.append(
            plsc.VectorSubcoreMesh(core_axis_name="c", subcore_axis_name="s"))
    return _mesh_cache[0]


def _pq_body(h_ref, wab_ref, p_ref, q_ref):
    pq = jnp.dot(h_ref[...], wab_ref[...], preferred_element_type=jnp.float32)
    p_ref[...] = pq[:, :D]
    q_ref[...] = pq[:, D:]


def _pq(h, wab):
    bn = 2000
    grid = (N // bn,)
    return pl.pallas_call(
        _pq_body,
        grid=grid,
        in_specs=[
            pl.BlockSpec((bn, D), lambda i: (i, 0)),
            pl.BlockSpec((D, 2 * D), lambda i: (0, 0)),
        ],
        out_specs=[
            pl.BlockSpec((bn, D), lambda i: (i, 0)),
            pl.BlockSpec((bn, D), lambda i: (i, 0)),
        ],
        out_shape=[
            jax.ShapeDtypeStruct((N, D), jnp.float32),
            jax.ShapeDtypeStruct((N, D), jnp.float32),
        ],
    )(h, wab)


def _gather(ES, C, p, q, x4flat, row_s, col_s, seq):
    EPS = ES // NW
    NCH = EPS // C

    def body(p_hbm, q_hbm, x4_hbm, row_hbm, col_hbm, seq_hbm,
             g_hbm, dx_hbm,
             idr, idc, sqg, pbuf, qbuf, x4v, dxbuf, g_sh, sem):
        cid = lax.axis_index("c")
        sid = lax.axis_index("s")
        base = (sid * NC + cid) * EPS
        sbase = sid * C
        pltpu.sync_copy(seq_hbm.at[pl.ds(sbase, C)], sqg)
        pltpu.sync_copy(x4_hbm, x4v)
        iota16 = lax.iota(jnp.int32, 16)
        lane = iota16 & 3
        pats = [u * 4 + (iota16 >> 2) for u in range(4)]

        @pl.loop(0, NCH)
        def _(k):
            off = base + k * C
            pltpu.sync_copy(row_hbm.at[pl.ds(off, C)], idr)
            pltpu.sync_copy(col_hbm.at[pl.ds(off, C)], idc)
            cp1 = pltpu.async_copy(p_hbm.at[idr], pbuf, sem)
            cp2 = pltpu.async_copy(q_hbm.at[idc], qbuf, sem)

            @pl.loop(0, C // 16)
            def _(g):
                r16 = idr[pl.ds(g * 16, 16)]
                c16 = idc[pl.ds(g * 16, 16)]
                for u in range(4):
                    pos_r = _dg(r16, pats[u]) * 4 + lane
                    pos_c = _dg(c16, pats[u]) * 4 + lane
                    xr = plsc.load_gather(x4v, [pos_r])
                    xc = plsc.load_gather(x4v, [pos_c])
                    dxbuf[pl.ds((g * 4 + u) * 16, 16)] = xr - xc

            pltpu.sync_copy(dxbuf, dx_hbm.at[pl.ds(off * 4, C * 4)])
            cp1.wait()
            pltpu.sync_copy(pbuf, g_sh.at[pl.ds(sbase, C)])
            cp2.wait()
            pltpu.sync_copy(qbuf, g_sh.at[sqg], add=True)
            pltpu.sync_copy(g_sh.at[pl.ds(sbase, C)], g_hbm.at[pl.ds(off, C)])

    f = pl.kernel(
        body,
        out_type=[
            jax.ShapeDtypeStruct((ES, D), jnp.float32),
            jax.ShapeDtypeStruct((ES * 4,), jnp.float32),
        ],
        mesh=_vector_mesh(),
        compiler_params=_sc_params(),
        scratch_types=[
            pltpu.VMEM((C,), jnp.int32),
            pltpu.VMEM((C,), jnp.int32),
            pltpu.VMEM((C,), jnp.int32),
            pltpu.VMEM((C, D), jnp.float32),
            pltpu.VMEM((C, D), jnp.float32),
            pltpu.VMEM((N * 4,), jnp.float32),
            pltpu.VMEM((C * 4,), jnp.float32),
            pltpu.VMEM_SHARED((NS * C, D), jnp.float32),
            pltpu.SemaphoreType.DMA,
        ],
    )
    return f(p, q, x4flat, row_s, col_s, seq)


def _edge_body(g_ref, dx_ref, ea_ref, wea_ref, wd_ref, be1_ref,
               we2_ref, be2_ref, wc1_ref, bc1_ref, wc2_ref, bc2_ref,
               m_ref, wdx_ref):
    dx = dx_ref[...]
    dist = jnp.sqrt(jnp.sum(dx * dx, axis=1, keepdims=True))
    z1 = (g_ref[...]
          + jnp.dot(ea_ref[...], wea_ref[...], preferred_element_type=jnp.float32)
          + dist * wd_ref[...]
          + be1_ref[...])
    a1 = z1 * jax.nn.sigmoid(z1)
    m = jnp.dot(a1, we2_ref[...], preferred_element_type=jnp.float32) + be2_ref[...]
    m_ref[...] = m
    z2 = jnp.dot(m, wc1_ref[...], preferred_element_type=jnp.float32) + bc1_ref[...]
    t = z2 * jax.nn.sigmoid(z2)
    cw = jnp.sum(t * wc2_ref[...], axis=1, keepdims=True) + bc2_ref[...]
    wdx_ref[...] = dx * cw


def _edge(ES, g, dx, ea, wea, wd, be1, we2, be2, wc1, bc1, wc2, bc2):
    be = 2000
    grid = (ES // be,)
    full = lambda i: (0, 0)
    return pl.pallas_call(
        _edge_body,
        grid=grid,
        in_specs=[
            pl.BlockSpec((be, D), lambda i: (i, 0)),
            pl.BlockSpec((be, 4), lambda i: (i, 0)),
            pl.BlockSpec((be, 16), lambda i: (i, 0)),
            pl.BlockSpec((16, D), full),
            pl.BlockSpec((1, D), full),
            pl.BlockSpec((1, D), full),
            pl.BlockSpec((D, D), full),
            pl.BlockSpec((1, D), full),
            pl.BlockSpec((D, D), full),
            pl.BlockSpec((1, D), full),
            pl.BlockSpec((1, D), full),
            pl.BlockSpec((1, 1), full),
        ],
        out_specs=[
            pl.BlockSpec((be, D), lambda i: (i, 0)),
            pl.BlockSpec((be, 4), lambda i: (i, 0)),
        ],
        out_shape=[
            jax.ShapeDtypeStruct((ES, D), jnp.float32),
            jax.ShapeDtypeStruct((ES, 4), jnp.float32),
        ],
    )(g, dx, ea, wea, wd, be1, we2, be2, wc1, bc1, wc2, bc2)


def _scatter(ES, C, m_ij, row_s, z128):
    EPS = ES // NW
    NCH = EPS // C

    def body(m_hbm, row_hbm, z128_hbm, mp_hbm, m_sh, idx, mbuf):
        cid = lax.axis_index("c")
        sid = lax.axis_index("s")
        base = (sid * NC + cid) * EPS
        stripe = sid * NSTRIPE
        pltpu.sync_copy(z128_hbm, m_sh.at[pl.ds(stripe, NSTRIPE)])
        plsc.subcore_barrier()

        @pl.loop(0, NCH)
        def _(k):
            off = base + k * C
            pltpu.sync_copy(row_hbm.at[pl.ds(off, C)], idx)
            pltpu.sync_copy(m_hbm.at[pl.ds(off, C)], mbuf)
            pltpu.sync_copy(mbuf, m_sh.at[idx], add=True)

        plsc.subcore_barrier()
        pltpu.sync_copy(m_sh.at[pl.ds(stripe, NSTRIPE)],
                        mp_hbm.at[cid, pl.ds(stripe, NSTRIPE)])

    f = pl.kernel(
        body,
        out_type=jax.ShapeDtypeStruct((NC, NP, D), jnp.float32),
        mesh=_vector_mesh(),
        scratch_types=[
            pltpu.VMEM_SHARED((NP, D), jnp.float32),
            pltpu.VMEM((C,), jnp.int32),
            pltpu.VMEM((C, D), jnp.float32),
        ],
    )
    return f(m_ij, row_s, z128)


def _cscatter(ES, C, w4flat, row_s):
    EPS = ES // NW
    NCH = EPS // C

    def body(w4_hbm, row_hbm, cp_hbm, idx, wvm, cacc):
        cid = lax.axis_index("c")
        sid = lax.axis_index("s")
        wid = sid * NC + cid
        base = wid * EPS
        zero16 = jnp.zeros((16,), jnp.float32)

        @pl.loop(0, (N * 4) // 16)
        def _(i):
            cacc[pl.ds(i * 16, 16)] = zero16

        iota16 = lax.iota(jnp.int32, 16)
        lane = iota16 & 3
        mask4 = iota16 < 4
        vpats = [u * 4 + lane for u in range(4)]

        @pl.loop(0, NCH)
        def _(k):
            off = base + k * C
            pltpu.sync_copy(row_hbm.at[pl.ds(off, C)], idx)
            pltpu.sync_copy(w4_hbm.at[pl.ds(off * 4, C * 4)], wvm)

            @pl.loop(0, C // 16)
            def _(g):
                rows16 = idx[pl.ds(g * 16, 16)]
                for t in range(4):
                    vload = wvm[pl.ds((g * 4 + t) * 16, 16)]
                    for u in range(4):
                        j = t * 4 + u
                        r = _dg(rows16, jnp.full((16,), j, jnp.int32))
                        v = _dg(vload, vpats[u])
                        plsc.addupdate_scatter(cacc, [r * 4 + lane], v,
                                               mask=mask4)

        pltpu.sync_copy(cacc, cp_hbm.at[wid])

    f = pl.kernel(
        body,
        out_type=jax.ShapeDtypeStruct((NW, N * 4), jnp.float32),
        mesh=_vector_mesh(),
        compiler_params=_sc_params(),
        scratch_types=[
            pltpu.VMEM((C,), jnp.int32),
            pltpu.VMEM((C * 4,), jnp.float32),
            pltpu.VMEM((N * 4,), jnp.float32),
        ],
    )
    return f(w4flat, row_s)


def _node_body(h_ref, m00_ref, m01_ref, m10_ref, m11_ref,
               a_ref, b_ref, bn1_ref, wn2_ref, bn2_ref, hn_ref):
    h = h_ref[...]
    mi = (m00_ref[...] + m01_ref[...]) + (m10_ref[...] + m11_ref[...])
    z = (jnp.dot(h, a_ref[...], preferred_element_type=jnp.float32)
         + jnp.dot(mi, b_ref[...], preferred_element_type=jnp.float32)
         + bn1_ref[...])
    u = z * jax.nn.sigmoid(z)
    hn_ref[...] = h + jnp.dot(u, wn2_ref[...], preferred_element_type=jnp.float32) + bn2_ref[...]


def _node(h, m00, m01, m10, m11, a, b, bn1, wn2, bn2):
    bn = 2000
    grid = (N // bn,)
    full = lambda i: (0, 0)
    blk = lambda i: (i, 0)
    return pl.pallas_call(
        _node_body,
        grid=grid,
        in_specs=[
            pl.BlockSpec((bn, D), blk),
            pl.BlockSpec((bn, D), blk),
            pl.BlockSpec((bn, D), blk),
            pl.BlockSpec((bn, D), blk),
            pl.BlockSpec((bn, D), blk),
            pl.BlockSpec((D, D), full),
            pl.BlockSpec((D, D), full),
            pl.BlockSpec((1, D), full),
            pl.BlockSpec((D, D), full),
            pl.BlockSpec((1, D), full),
        ],
        out_specs=pl.BlockSpec((bn, D), blk),
        out_shape=jax.ShapeDtypeStruct((N, D), jnp.float32),
    )(h, m00, m01, m10, m11, a, b, bn1, wn2, bn2)


def _coord_body(x4_ref, cp0_ref, cp1_ref, xn_ref):
    xn_ref[...] = (x4_ref[...] + jnp.sum(cp0_ref[...], axis=0)
                   + jnp.sum(cp1_ref[...], axis=0))


def _coord(x4flat, cp0, cp1):
    return pl.pallas_call(
        _coord_body,
        grid=(1,),
        in_specs=[
            pl.BlockSpec((N * 4,), lambda i: (0,)),
            pl.BlockSpec((NW, N * 4), lambda i: (0, 0)),
            pl.BlockSpec((NW, N * 4), lambda i: (0, 0)),
        ],
        out_specs=pl.BlockSpec((N * 4,), lambda i: (0,)),
        out_shape=jax.ShapeDtypeStruct((N * 4,), jnp.float32),
    )(x4flat, cp0, cp1)


def kernel(h, x, edge_index, edge_attr,
           W_e1, b_e1, W_e2, b_e2,
           W_n1, b_n1, W_n2, b_n2,
           W_c1, b_c1, W_c2, b_c2):
    row = edge_index[0]
    col = edge_index[1]
    x4flat = jnp.pad(x, ((0, 0), (0, 1))).reshape(N * 4)
    wab = jnp.concatenate([W_e1[:D], W_e1[D:2 * D]], axis=1)
    wea = W_e1[2 * D:2 * D + 16]
    wd = W_e1[2 * D + 16:].reshape(1, D)
    z128 = jnp.zeros((NSTRIPE, D), jnp.float32)
    seq1 = jnp.arange(NS * CS1, dtype=jnp.int32)
    seq2 = jnp.arange(NS * CS2, dtype=jnp.int32)

    p, q = _pq(h, wab)

    ew = (wea, wd, b_e1.reshape(1, D), W_e2, b_e2.reshape(1, D),
          W_c1, b_c1.reshape(1, D), W_c2.reshape(1, D), b_c2.reshape(1, 1))

    g1, dxf1 = _gather(ES1, CS1, p, q, x4flat, row[:ES1], col[:ES1], seq1)
    g2, dxf2 = _gather(ES2, CS2, p, q, x4flat, row[ES1:], col[ES1:], seq2)
    m1, w1 = _edge(ES1, g1, dxf1.reshape(ES1, 4), edge_attr[:ES1], *ew)
    m2, w2 = _edge(ES2, g2, dxf2.reshape(ES2, 4), edge_attr[ES1:], *ew)
    mp1 = _scatter(ES1, CS1, m1, row[:ES1], z128)
    mp2 = _scatter(ES2, CS2, m2, row[ES1:], z128)
    cp1 = _cscatter(ES1, CS1, w1.reshape(ES1 * 4), row[:ES1])
    cp2 = _cscatter(ES2, CS2, w2.reshape(ES2 * 4), row[ES1:])

    hn = _node(h, mp1[0, :N], mp1[1, :N], mp2[0, :N], mp2[1, :N],
               W_n1[:D], W_n1[D:], b_n1.reshape(1, D),
               W_n2, b_n2.reshape(1, D))
    xn = _coord(x4flat, cp1, cp2)
    return (hn, xn.reshape(N, 4)[:, :3])


# speedup vs baseline: 1.3086x; 1.2334x over previous
import dataclasses
import functools

import jax
import jax.numpy as jnp
from jax import lax
from jax.experimental import pallas as pl
from jax.experimental.pallas import tpu as pltpu
from jax.experimental.pallas import tpu_sc as plsc

N = 10000
E = 320000
D = 128
NC = 2
NS = 16
NW = NC * NS
NP = 10240
NSTRIPE = NP // NS
ES1 = 192000
ES2 = E - ES1
CS1 = 48
CS2 = 32


def _sc_params():
    cp = pltpu.CompilerParams()
    if "needs_layout_passes" in pltpu.CompilerParams.__dataclass_fields__:
        cp = dataclasses.replace(cp, needs_layout_passes=False)
    return cp


def _dg(v, idx16):
    return lax.gather(
        v, idx16[:, None],
        lax.GatherDimensionNumbers(offset_dims=(), collapsed_slice_dims=(0,),
                                   start_index_map=(0,)),
        (1,), mode=lax.GatherScatterMode.PROMISE_IN_BOUNDS)


_mesh_cache = []


def _vector_mesh():
    if not _mesh_cache:
        _mesh_cache.append(
            plsc.VectorSubcoreMesh(core_axis_name="c", subcore_axis_name="s"))
    return _mesh_cache[0]


def _pq_body(h_ref, wab_ref, p_ref, q_ref):
    pq = jnp.dot(h_ref[...], wab_ref[...], preferred_element_type=jnp.float32)
    p_ref[...] = pq[:, :D]
    q_ref[...] = pq[:, D:]


def _pq(h, wab):
    bn = 2000
    grid = (N // bn,)
    return pl.pallas_call(
        _pq_body,
        grid=grid,
        in_specs=[
            pl.BlockSpec((bn, D), lambda i: (i, 0)),
            pl.BlockSpec((D, 2 * D), lambda i: (0, 0)),
        ],
        out_specs=[
            pl.BlockSpec((bn, D), lambda i: (i, 0)),
            pl.BlockSpec((bn, D), lambda i: (i, 0)),
        ],
        out_shape=[
            jax.ShapeDtypeStruct((N, D), jnp.float32),
            jax.ShapeDtypeStruct((N, D), jnp.float32),
        ],
    )(h, wab)


def _gather(ES, C, p, q, x4flat, row_s, col_s, seq):
    EPS = ES // NW
    NCH = EPS // C

    def body(p_hbm, q_hbm, x4_hbm, row_hbm, col_hbm, seq_hbm,
             g_hbm, dx_hbm,
             idr0, idr1, idc0, idc1, sqg0, sqg1,
             pbuf0, pbuf1, qbuf0, qbuf1, x4v, dxbuf0, dxbuf1,
             g_sh, psem0, psem1, qsem0, qsem1):
        cid = lax.axis_index("c")
        sid = lax.axis_index("s")
        base = (sid * NC + cid) * EPS
        idr = (idr0, idr1)
        idc = (idc0, idc1)
        sqg = (sqg0, sqg1)
        pbuf = (pbuf0, pbuf1)
        qbuf = (qbuf0, qbuf1)
        dxbuf = (dxbuf0, dxbuf1)
        psem = (psem0, psem1)
        qsem = (qsem0, qsem1)
        sreg = (sid * 2 * C, sid * 2 * C + C)
        pltpu.sync_copy(seq_hbm.at[pl.ds(sreg[0], C)], sqg0)
        pltpu.sync_copy(seq_hbm.at[pl.ds(sreg[1], C)], sqg1)
        pltpu.sync_copy(x4_hbm, x4v)
        iota16 = lax.iota(jnp.int32, 16)
        lane = iota16 & 3
        pats = [u * 4 + (iota16 >> 2) for u in range(4)]

        def load_idx(k, b):
            off = base + k * C
            pltpu.sync_copy(row_hbm.at[pl.ds(off, C)], idr[b])
            pltpu.sync_copy(col_hbm.at[pl.ds(off, C)], idc[b])

        def issue(b):
            pltpu.async_copy(p_hbm.at[idr[b]], pbuf[b], psem[b])
            pltpu.async_copy(q_hbm.at[idc[b]], qbuf[b], qsem[b])

        def process(k, b):
            off = base + k * C

            @pl.loop(0, C // 16)
            def _(g):
                r16 = idr[b][pl.ds(g * 16, 16)]
                c16 = idc[b][pl.ds(g * 16, 16)]
                for u in range(4):
                    pos_r = _dg(r16, pats[u]) * 4 + lane
                    pos_c = _dg(c16, pats[u]) * 4 + lane
                    xr = plsc.load_gather(x4v, [pos_r])
                    xc = plsc.load_gather(x4v, [pos_c])
                    dxbuf[b][pl.ds((g * 4 + u) * 16, 16)] = xr - xc

            pltpu.sync_copy(dxbuf[b], dx_hbm.at[pl.ds(off * 4, C * 4)])
            pltpu.make_async_copy(p_hbm.at[idr[b]], pbuf[b], psem[b]).wait()
            pltpu.sync_copy(pbuf[b], g_sh.at[pl.ds(sreg[b], C)])
            pltpu.make_async_copy(q_hbm.at[idc[b]], qbuf[b], qsem[b]).wait()
            pltpu.sync_copy(qbuf[b], g_sh.at[sqg[b]], add=True)
            pltpu.sync_copy(g_sh.at[pl.ds(sreg[b], C)], g_hbm.at[pl.ds(off, C)])

        load_idx(0, 0)
        issue(0)

        @pl.loop(0, (NCH - 1) // 2)
        def _(j):
            for b in (0, 1):
                k = 2 * j + b
                load_idx(k + 1, 1 - b)
                issue(1 - b)
                process(k, b)

        process(NCH - 1, (NCH - 1) % 2)

    f = pl.kernel(
        body,
        out_type=[
            jax.ShapeDtypeStruct((ES, D), jnp.float32),
            jax.ShapeDtypeStruct((ES * 4,), jnp.float32),
        ],
        mesh=_vector_mesh(),
        compiler_params=_sc_params(),
        scratch_types=[
            pltpu.VMEM((C,), jnp.int32),
            pltpu.VMEM((C,), jnp.int32),
            pltpu.VMEM((C,), jnp.int32),
            pltpu.VMEM((C,), jnp.int32),
            pltpu.VMEM((C,), jnp.int32),
            pltpu.VMEM((C,), jnp.int32),
            pltpu.VMEM((C, D), jnp.float32),
            pltpu.VMEM((C, D), jnp.float32),
            pltpu.VMEM((C, D), jnp.float32),
            pltpu.VMEM((C, D), jnp.float32),
            pltpu.VMEM((N * 4,), jnp.float32),
            pltpu.VMEM((C * 4,), jnp.float32),
            pltpu.VMEM((C * 4,), jnp.float32),
            pltpu.VMEM_SHARED((NS * 2 * C, D), jnp.float32),
            pltpu.SemaphoreType.DMA,
            pltpu.SemaphoreType.DMA,
            pltpu.SemaphoreType.DMA,
            pltpu.SemaphoreType.DMA,
        ],
    )
    return f(p, q, x4flat, row_s, col_s, seq)


def _edge_body(g_ref, dx_ref, ea_ref, wea_ref, wd_ref, be1_ref,
               we2_ref, be2_ref, wc1_ref, bc1_ref, wc2_ref, bc2_ref,
               m_ref, wdx_ref):
    dx = dx_ref[...]
    dist = jnp.sqrt(jnp.sum(dx * dx, axis=1, keepdims=True))
    z1 = (g_ref[...]
          + jnp.dot(ea_ref[...], wea_ref[...], preferred_element_type=jnp.float32)
          + dist * wd_ref[...]
          + be1_ref[...])
    a1 = z1 * jax.nn.sigmoid(z1)
    m = jnp.dot(a1, we2_ref[...], preferred_element_type=jnp.float32) + be2_ref[...]
    m_ref[...] = m
    z2 = jnp.dot(m, wc1_ref[...], preferred_element_type=jnp.float32) + bc1_ref[...]
    t = z2 * jax.nn.sigmoid(z2)
    cw = jnp.sum(t * wc2_ref[...], axis=1, keepdims=True) + bc2_ref[...]
    wdx_ref[...] = dx * cw


def _edge(ES, g, dx, ea, wea, wd, be1, we2, be2, wc1, bc1, wc2, bc2):
    be = 2000
    grid = (ES // be,)
    full = lambda i: (0, 0)
    return pl.pallas_call(
        _edge_body,
        grid=grid,
        in_specs=[
            pl.BlockSpec((be, D), lambda i: (i, 0)),
            pl.BlockSpec((be, 4), lambda i: (i, 0)),
            pl.BlockSpec((be, 16), lambda i: (i, 0)),
            pl.BlockSpec((16, D), full),
            pl.BlockSpec((1, D), full),
            pl.BlockSpec((1, D), full),
            pl.BlockSpec((D, D), full),
            pl.BlockSpec((1, D), full),
            pl.BlockSpec((D, D), full),
            pl.BlockSpec((1, D), full),
            pl.BlockSpec((1, D), full),
            pl.BlockSpec((1, 1), full),
        ],
        out_specs=[
            pl.BlockSpec((be, D), lambda i: (i, 0)),
            pl.BlockSpec((be, 4), lambda i: (i, 0)),
        ],
        out_shape=[
            jax.ShapeDtypeStruct((ES, D), jnp.float32),
            jax.ShapeDtypeStruct((ES, 4), jnp.float32),
        ],
    )(g, dx, ea, wea, wd, be1, we2, be2, wc1, bc1, wc2, bc2)


def _scatter(ES, C, m_ij, row_s, z128):
    EPS = ES // NW
    NCH = EPS // C

    def body(m_hbm, row_hbm, z128_hbm, mp_hbm, m_sh,
             idx0, idx1, mbuf0, mbuf1, msem0, msem1):
        cid = lax.axis_index("c")
        sid = lax.axis_index("s")
        base = (sid * NC + cid) * EPS
        stripe = sid * NSTRIPE
        idx = (idx0, idx1)
        mbuf = (mbuf0, mbuf1)
        msem = (msem0, msem1)
        pltpu.sync_copy(z128_hbm, m_sh.at[pl.ds(stripe, NSTRIPE)])
        plsc.subcore_barrier()

        def load(k, b):
            off = base + k * C
            pltpu.sync_copy(row_hbm.at[pl.ds(off, C)], idx[b])
            pltpu.async_copy(m_hbm.at[pl.ds(off, C)], mbuf[b], msem[b])

        def process(k, b):
            off = base + k * C
            pltpu.make_async_copy(m_hbm.at[pl.ds(off, C)], mbuf[b],
                                  msem[b]).wait()
            pltpu.sync_copy(mbuf[b], m_sh.at[idx[b]], add=True)

        load(0, 0)

        @pl.loop(0, (NCH - 1) // 2)
        def _(j):
            for b in (0, 1):
                k = 2 * j + b
                load(k + 1, 1 - b)
                process(k, b)

        process(NCH - 1, (NCH - 1) % 2)
        plsc.subcore_barrier()
        pltpu.sync_copy(m_sh.at[pl.ds(stripe, NSTRIPE)],
                        mp_hbm.at[cid, pl.ds(stripe, NSTRIPE)])

    f = pl.kernel(
        body,
        out_type=jax.ShapeDtypeStruct((NC, NP, D), jnp.float32),
        mesh=_vector_mesh(),
        scratch_types=[
            pltpu.VMEM_SHARED((NP, D), jnp.float32),
            pltpu.VMEM((C,), jnp.int32),
            pltpu.VMEM((C,), jnp.int32),
            pltpu.VMEM((C, D), jnp.float32),
            pltpu.VMEM((C, D), jnp.float32),
            pltpu.SemaphoreType.DMA,
            pltpu.SemaphoreType.DMA,
        ],
    )
    return f(m_ij, row_s, z128)


def _cscatter(ES, C, w4flat, row_s):
    EPS = ES // NW
    NCH = EPS // C

    def body(w4_hbm, row_hbm, cp_hbm, idx0, idx1, wvm0, wvm1, cacc,
             wsem0, wsem1):
        cid = lax.axis_index("c")
        sid = lax.axis_index("s")
        wid = sid * NC + cid
        base = wid * EPS
        idx = (idx0, idx1)
        wvm = (wvm0, wvm1)
        wsem = (wsem0, wsem1)
        zero16 = jnp.zeros((16,), jnp.float32)

        @pl.loop(0, (N * 4) // 16)
        def _(i):
            cacc[pl.ds(i * 16, 16)] = zero16

        iota16 = lax.iota(jnp.int32, 16)
        lane = iota16 & 3
        mask4 = iota16 < 4
        vpats = [u * 4 + lane for u in range(4)]

        def load(k, b):
            off = base + k * C
            pltpu.sync_copy(row_hbm.at[pl.ds(off, C)], idx[b])
            pltpu.async_copy(w4_hbm.at[pl.ds(off * 4, C * 4)], wvm[b],
                             wsem[b])

        def process(k, b):
            off = base + k * C
            pltpu.make_async_copy(w4_hbm.at[pl.ds(off * 4, C * 4)], wvm[b],
                                  wsem[b]).wait()

            @pl.loop(0, C // 16)
            def _(g):
                rows16 = idx[b][pl.ds(g * 16, 16)]
                for t in range(4):
                    vload = wvm[b][pl.ds((g * 4 + t) * 16, 16)]
                    for u in range(4):
                        j2 = t * 4 + u
                        r = _dg(rows16, jnp.full((16,), j2, jnp.int32))
                        v = _dg(vload, vpats[u])
                        plsc.addupdate_scatter(cacc, [r * 4 + lane], v,
                                               mask=mask4)

        load(0, 0)

        @pl.loop(0, (NCH - 1) // 2)
        def _(j):
            for b in (0, 1):
                k = 2 * j + b
                load(k + 1, 1 - b)
                process(k, b)

        process(NCH - 1, (NCH - 1) % 2)
        pltpu.sync_copy(cacc, cp_hbm.at[wid])

    f = pl.kernel(
        body,
        out_type=jax.ShapeDtypeStruct((NW, N * 4), jnp.float32),
        mesh=_vector_mesh(),
        compiler_params=_sc_params(),
        scratch_types=[
            pltpu.VMEM((C,), jnp.int32),
            pltpu.VMEM((C,), jnp.int32),
            pltpu.VMEM((C * 4,), jnp.float32),
            pltpu.VMEM((C * 4,), jnp.float32),
            pltpu.VMEM((N * 4,), jnp.float32),
            pltpu.SemaphoreType.DMA,
            pltpu.SemaphoreType.DMA,
        ],
    )
    return f(w4flat, row_s)


def _node_body(h_ref, m00_ref, m01_ref, m10_ref, m11_ref,
               a_ref, b_ref, bn1_ref, wn2_ref, bn2_ref, hn_ref):
    h = h_ref[...]
    mi = (m00_ref[...] + m01_ref[...]) + (m10_ref[...] + m11_ref[...])
    z = (jnp.dot(h, a_ref[...], preferred_element_type=jnp.float32)
         + jnp.dot(mi, b_ref[...], preferred_element_type=jnp.float32)
         + bn1_ref[...])
    u = z * jax.nn.sigmoid(z)
    hn_ref[...] = h + jnp.dot(u, wn2_ref[...], preferred_element_type=jnp.float32) + bn2_ref[...]


def _node(h, m00, m01, m10, m11, a, b, bn1, wn2, bn2):
    bn = 2000
    grid = (N // bn,)
    full = lambda i: (0, 0)
    blk = lambda i: (i, 0)
    return pl.pallas_call(
        _node_body,
        grid=grid,
        in_specs=[
            pl.BlockSpec((bn, D), blk),
            pl.BlockSpec((bn, D), blk),
            pl.BlockSpec((bn, D), blk),
            pl.BlockSpec((bn, D), blk),
            pl.BlockSpec((bn, D), blk),
            pl.BlockSpec((D, D), full),
            pl.BlockSpec((D, D), full),
            pl.BlockSpec((1, D), full),
            pl.BlockSpec((D, D), full),
            pl.BlockSpec((1, D), full),
        ],
        out_specs=pl.BlockSpec((bn, D), blk),
        out_shape=jax.ShapeDtypeStruct((N, D), jnp.float32),
    )(h, m00, m01, m10, m11, a, b, bn1, wn2, bn2)


def _coord_body(x4_ref, cp0_ref, cp1_ref, xn_ref):
    xn_ref[...] = (x4_ref[...] + jnp.sum(cp0_ref[...], axis=0)
                   + jnp.sum(cp1_ref[...], axis=0))


def _coord(x4flat, cp0, cp1):
    return pl.pallas_call(
        _coord_body,
        grid=(1,),
        in_specs=[
            pl.BlockSpec((N * 4,), lambda i: (0,)),
            pl.BlockSpec((NW, N * 4), lambda i: (0, 0)),
            pl.BlockSpec((NW, N * 4), lambda i: (0, 0)),
        ],
        out_specs=pl.BlockSpec((N * 4,), lambda i: (0,)),
        out_shape=jax.ShapeDtypeStruct((N * 4,), jnp.float32),
    )(x4flat, cp0, cp1)


def kernel(h, x, edge_index, edge_attr,
           W_e1, b_e1, W_e2, b_e2,
           W_n1, b_n1, W_n2, b_n2,
           W_c1, b_c1, W_c2, b_c2):
    row = edge_index[0]
    col = edge_index[1]
    x4flat = jnp.pad(x, ((0, 0), (0, 1))).reshape(N * 4)
    wab = jnp.concatenate([W_e1[:D], W_e1[D:2 * D]], axis=1)
    wea = W_e1[2 * D:2 * D + 16]
    wd = W_e1[2 * D + 16:].reshape(1, D)
    z128 = jnp.zeros((NSTRIPE, D), jnp.float32)
    seq1 = jnp.arange(NS * 2 * CS1, dtype=jnp.int32)
    seq2 = jnp.arange(NS * 2 * CS2, dtype=jnp.int32)

    p, q = _pq(h, wab)

    ew = (wea, wd, b_e1.reshape(1, D), W_e2, b_e2.reshape(1, D),
          W_c1, b_c1.reshape(1, D), W_c2.reshape(1, D), b_c2.reshape(1, 1))

    g1, dxf1 = _gather(ES1, CS1, p, q, x4flat, row[:ES1], col[:ES1], seq1)
    g2, dxf2 = _gather(ES2, CS2, p, q, x4flat, row[ES1:], col[ES1:], seq2)
    m1, w1 = _edge(ES1, g1, dxf1.reshape(ES1, 4), edge_attr[:ES1], *ew)
    m2, w2 = _edge(ES2, g2, dxf2.reshape(ES2, 4), edge_attr[ES1:], *ew)
    mp1 = _scatter(ES1, CS1, m1, row[:ES1], z128)
    mp2 = _scatter(ES2, CS2, m2, row[ES1:], z128)
    cp1 = _cscatter(ES1, CS1, w1.reshape(ES1 * 4), row[:ES1])
    cp2 = _cscatter(ES2, CS2, w2.reshape(ES2 * 4), row[ES1:])

    hn = _node(h, mp1[0, :N], mp1[1, :N], mp2[0, :N], mp2[1, :N],
               W_n1[:D], W_n1[D:], b_n1.reshape(1, D),
               W_n2, b_n2.reshape(1, D))
    xn = _coord(x4flat, cp1, cp2)
    return (hn, xn.reshape(N, 4)[:, :3])


# speedup vs baseline: 1.3816x; 1.0557x over previous
import dataclasses
import functools

import jax
import jax.numpy as jnp
from jax import lax
from jax.experimental import pallas as pl
from jax.experimental.pallas import tpu as pltpu
from jax.experimental.pallas import tpu_sc as plsc

N = 10000
E = 320000
D = 128
NC = 2
NS = 16
NW = NC * NS
NP = 10240
NSTRIPE = NP // NS
ES1 = 192000
ES2 = E - ES1
CS1 = 48
CS2 = 32


def _sc_params():
    cp = pltpu.CompilerParams()
    if "needs_layout_passes" in pltpu.CompilerParams.__dataclass_fields__:
        cp = dataclasses.replace(cp, needs_layout_passes=False)
    return cp


def _dg(v, idx16):
    return lax.gather(
        v, idx16[:, None],
        lax.GatherDimensionNumbers(offset_dims=(), collapsed_slice_dims=(0,),
                                   start_index_map=(0,)),
        (1,), mode=lax.GatherScatterMode.PROMISE_IN_BOUNDS)


_mesh_cache = []


def _vector_mesh():
    if not _mesh_cache:
        _mesh_cache.append(
            plsc.VectorSubcoreMesh(core_axis_name="c", subcore_axis_name="s"))
    return _mesh_cache[0]


def _pq_body(h_ref, wab_ref, p_ref, q_ref):
    pq = jnp.dot(h_ref[...], wab_ref[...], preferred_element_type=jnp.float32)
    p_ref[...] = pq[:, :D]
    q_ref[...] = pq[:, D:]


def _pq(h, wab):
    bn = 2000
    grid = (N // bn,)
    return pl.pallas_call(
        _pq_body,
        grid=grid,
        in_specs=[
            pl.BlockSpec((bn, D), lambda i: (i, 0)),
            pl.BlockSpec((D, 2 * D), lambda i: (0, 0)),
        ],
        out_specs=[
            pl.BlockSpec((bn, D), lambda i: (i, 0)),
            pl.BlockSpec((bn, D), lambda i: (i, 0)),
        ],
        out_shape=[
            jax.ShapeDtypeStruct((N, D), jnp.float32),
            jax.ShapeDtypeStruct((N, D), jnp.float32),
        ],
    )(h, wab)


def _gather(ES, C, p, q, x4flat, row_s, col_s, seq):
    EPS = ES // NW
    NCH = EPS // C

    def body(p_hbm, q_hbm, x4_hbm, row_hbm, col_hbm, seq_hbm,
             g_hbm, dx_hbm,
             idr0, idr1, idc0, idc1, sqg0, sqg1,
             pbuf0, pbuf1, qbuf0, qbuf1, x4v, dxbuf0, dxbuf1,
             g_sh, psem0, psem1, qsem0, qsem1,
             dxsem0, dxsem1, wbsem0, wbsem1):
        cid = lax.axis_index("c")
        sid = lax.axis_index("s")
        base = (sid * NC + cid) * EPS
        idr = (idr0, idr1)
        idc = (idc0, idc1)
        sqg = (sqg0, sqg1)
        pbuf = (pbuf0, pbuf1)
        qbuf = (qbuf0, qbuf1)
        dxbuf = (dxbuf0, dxbuf1)
        psem = (psem0, psem1)
        qsem = (qsem0, qsem1)
        dxsem = (dxsem0, dxsem1)
        wbsem = (wbsem0, wbsem1)
        sreg = (sid * 2 * C, sid * 2 * C + C)
        pltpu.sync_copy(seq_hbm.at[pl.ds(sreg[0], C)], sqg0)
        pltpu.sync_copy(seq_hbm.at[pl.ds(sreg[1], C)], sqg1)
        pltpu.sync_copy(x4_hbm, x4v)
        iota16 = lax.iota(jnp.int32, 16)
        lane = iota16 & 3
        pats = [u * 4 + (iota16 >> 2) for u in range(4)]

        def load_idx(k, b):
            off = base + k * C
            pltpu.sync_copy(row_hbm.at[pl.ds(off, C)], idr[b])
            pltpu.sync_copy(col_hbm.at[pl.ds(off, C)], idc[b])

        def issue(b):
            pltpu.async_copy(p_hbm.at[idr[b]], pbuf[b], psem[b])
            pltpu.async_copy(q_hbm.at[idc[b]], qbuf[b], qsem[b])

        def process(k, b):
            off = base + k * C
            offp = off - 2 * C

            @pl.when(k >= 2)
            def _():
                pltpu.make_async_copy(
                    dxbuf[b], dx_hbm.at[pl.ds(offp * 4, C * 4)],
                    dxsem[b]).wait()
                pltpu.make_async_copy(
                    g_sh.at[pl.ds(sreg[b], C)], g_hbm.at[pl.ds(offp, C)],
                    wbsem[b]).wait()

            @pl.loop(0, C // 16)
            def _(g):
                r16 = idr[b][pl.ds(g * 16, 16)]
                c16 = idc[b][pl.ds(g * 16, 16)]
                for u in range(4):
                    pos_r = _dg(r16, pats[u]) * 4 + lane
                    pos_c = _dg(c16, pats[u]) * 4 + lane
                    xr = plsc.load_gather(x4v, [pos_r])
                    xc = plsc.load_gather(x4v, [pos_c])
                    dxbuf[b][pl.ds((g * 4 + u) * 16, 16)] = xr - xc

            pltpu.async_copy(dxbuf[b], dx_hbm.at[pl.ds(off * 4, C * 4)],
                             dxsem[b])
            pltpu.make_async_copy(p_hbm.at[idr[b]], pbuf[b], psem[b]).wait()
            pltpu.sync_copy(pbuf[b], g_sh.at[pl.ds(sreg[b], C)])
            pltpu.make_async_copy(q_hbm.at[idc[b]], qbuf[b], qsem[b]).wait()
            pltpu.sync_copy(qbuf[b], g_sh.at[sqg[b]], add=True)
            pltpu.async_copy(g_sh.at[pl.ds(sreg[b], C)],
                             g_hbm.at[pl.ds(off, C)], wbsem[b])

        load_idx(0, 0)
        issue(0)

        @pl.loop(0, (NCH - 1) // 2)
        def _(j):
            for b in (0, 1):
                k = 2 * j + b
                load_idx(k + 1, 1 - b)
                issue(1 - b)
                process(k, b)

        process(NCH - 1, (NCH - 1) % 2)
        for kk in (NCH - 2, NCH - 1):
            bb = kk % 2
            offk = base + kk * C
            pltpu.make_async_copy(
                dxbuf[bb], dx_hbm.at[pl.ds(offk * 4, C * 4)],
                dxsem[bb]).wait()
            pltpu.make_async_copy(
                g_sh.at[pl.ds(sreg[bb], C)], g_hbm.at[pl.ds(offk, C)],
                wbsem[bb]).wait()

    f = pl.kernel(
        body,
        out_type=[
            jax.ShapeDtypeStruct((ES, D), jnp.float32),
            jax.ShapeDtypeStruct((ES * 4,), jnp.float32),
        ],
        mesh=_vector_mesh(),
        compiler_params=_sc_params(),
        scratch_types=[
            pltpu.VMEM((C,), jnp.int32),
            pltpu.VMEM((C,), jnp.int32),
            pltpu.VMEM((C,), jnp.int32),
            pltpu.VMEM((C,), jnp.int32),
            pltpu.VMEM((C,), jnp.int32),
            pltpu.VMEM((C,), jnp.int32),
            pltpu.VMEM((C, D), jnp.float32),
            pltpu.VMEM((C, D), jnp.float32),
            pltpu.VMEM((C, D), jnp.float32),
            pltpu.VMEM((C, D), jnp.float32),
            pltpu.VMEM((N * 4,), jnp.float32),
            pltpu.VMEM((C * 4,), jnp.float32),
            pltpu.VMEM((C * 4,), jnp.float32),
            pltpu.VMEM_SHARED((NS * 2 * C, D), jnp.float32),
            pltpu.SemaphoreType.DMA,
            pltpu.SemaphoreType.DMA,
            pltpu.SemaphoreType.DMA,
            pltpu.SemaphoreType.DMA,
            pltpu.SemaphoreType.DMA,
            pltpu.SemaphoreType.DMA,
            pltpu.SemaphoreType.DMA,
            pltpu.SemaphoreType.DMA,
        ],
    )
    return f(p, q, x4flat, row_s, col_s, seq)


def _edge_body(g_ref, dx_ref, ea_ref, wea_ref, wd_ref, be1_ref,
               we2_ref, be2_ref, wc1_ref, bc1_ref, wc2_ref, bc2_ref,
               m_ref, wdx_ref):
    dx = dx_ref[...]
    dist = jnp.sqrt(jnp.sum(dx * dx, axis=1, keepdims=True))
    z1 = (g_ref[...]
          + jnp.dot(ea_ref[...], wea_ref[...], preferred_element_type=jnp.float32)
          + dist * wd_ref[...]
          + be1_ref[...])
    a1 = z1 * jax.nn.sigmoid(z1)
    m = jnp.dot(a1, we2_ref[...], preferred_element_type=jnp.float32) + be2_ref[...]
    m_ref[...] = m
    z2 = jnp.dot(m, wc1_ref[...], preferred_element_type=jnp.float32) + bc1_ref[...]
    t = z2 * jax.nn.sigmoid(z2)
    cw = jnp.sum(t * wc2_ref[...], axis=1, keepdims=True) + bc2_ref[...]
    wdx_ref[...] = dx * cw


def _edge(ES, g, dx, ea, wea, wd, be1, we2, be2, wc1, bc1, wc2, bc2):
    be = 2000
    grid = (ES // be,)
    full = lambda i: (0, 0)
    return pl.pallas_call(
        _edge_body,
        grid=grid,
        in_specs=[
            pl.BlockSpec((be, D), lambda i: (i, 0)),
            pl.BlockSpec((be, 4), lambda i: (i, 0)),
            pl.BlockSpec((be, 16), lambda i: (i, 0)),
            pl.BlockSpec((16, D), full),
            pl.BlockSpec((1, D), full),
            pl.BlockSpec((1, D), full),
            pl.BlockSpec((D, D), full),
            pl.BlockSpec((1, D), full),
            pl.BlockSpec((D, D), full),
            pl.BlockSpec((1, D), full),
            pl.BlockSpec((1, D), full),
            pl.BlockSpec((1, 1), full),
        ],
        out_specs=[
            pl.BlockSpec((be, D), lambda i: (i, 0)),
            pl.BlockSpec((be, 4), lambda i: (i, 0)),
        ],
        out_shape=[
            jax.ShapeDtypeStruct((ES, D), jnp.float32),
            jax.ShapeDtypeStruct((ES, 4), jnp.float32),
        ],
    )(g, dx, ea, wea, wd, be1, we2, be2, wc1, bc1, wc2, bc2)


def _scatter(ES, C, m_ij, row_s, z128):
    EPS = ES // NW
    NCH = EPS // C

    def body(m_hbm, row_hbm, z128_hbm, mp_hbm, m_sh,
             idx0, idx1, mbuf0, mbuf1, msem0, msem1):
        cid = lax.axis_index("c")
        sid = lax.axis_index("s")
        base = (sid * NC + cid) * EPS
        stripe = sid * NSTRIPE
        idx = (idx0, idx1)
        mbuf = (mbuf0, mbuf1)
        msem = (msem0, msem1)
        pltpu.sync_copy(z128_hbm, m_sh.at[pl.ds(stripe, NSTRIPE)])
        plsc.subcore_barrier()

        def load(k, b):
            off = base + k * C
            pltpu.sync_copy(row_hbm.at[pl.ds(off, C)], idx[b])
            pltpu.async_copy(m_hbm.at[pl.ds(off, C)], mbuf[b], msem[b])

        def process(k, b):
            off = base + k * C
            pltpu.make_async_copy(m_hbm.at[pl.ds(off, C)], mbuf[b],
                                  msem[b]).wait()
            pltpu.sync_copy(mbuf[b], m_sh.at[idx[b]], add=True)

        load(0, 0)

        @pl.loop(0, (NCH - 1) // 2)
        def _(j):
            for b in (0, 1):
                k = 2 * j + b
                load(k + 1, 1 - b)
                process(k, b)

        process(NCH - 1, (NCH - 1) % 2)
        plsc.subcore_barrier()
        pltpu.sync_copy(m_sh.at[pl.ds(stripe, NSTRIPE)],
                        mp_hbm.at[cid, pl.ds(stripe, NSTRIPE)])

    f = pl.kernel(
        body,
        out_type=jax.ShapeDtypeStruct((NC, NP, D), jnp.float32),
        mesh=_vector_mesh(),
        scratch_types=[
            pltpu.VMEM_SHARED((NP, D), jnp.float32),
            pltpu.VMEM((C,), jnp.int32),
            pltpu.VMEM((C,), jnp.int32),
            pltpu.VMEM((C, D), jnp.float32),
            pltpu.VMEM((C, D), jnp.float32),
            pltpu.SemaphoreType.DMA,
            pltpu.SemaphoreType.DMA,
        ],
    )
    return f(m_ij, row_s, z128)


def _cscatter(ES, C, w4flat, row_s):
    EPS = ES // NW
    NCH = EPS // C

    def body(w4_hbm, row_hbm, cp_hbm, idx0, idx1, wvm0, wvm1, cacc,
             wsem0, wsem1):
        cid = lax.axis_index("c")
        sid = lax.axis_index("s")
        wid = sid * NC + cid
        base = wid * EPS
        idx = (idx0, idx1)
        wvm = (wvm0, wvm1)
        wsem = (wsem0, wsem1)
        zero16 = jnp.zeros((16,), jnp.float32)

        @pl.loop(0, (N * 4) // 16)
        def _(i):
            cacc[pl.ds(i * 16, 16)] = zero16

        iota16 = lax.iota(jnp.int32, 16)
        lane = iota16 & 3
        mask4 = iota16 < 4
        vpats = [u * 4 + lane for u in range(4)]

        def load(k, b):
            off = base + k * C
            pltpu.sync_copy(row_hbm.at[pl.ds(off, C)], idx[b])
            pltpu.async_copy(w4_hbm.at[pl.ds(off * 4, C * 4)], wvm[b],
                             wsem[b])

        def process(k, b):
            off = base + k * C
            pltpu.make_async_copy(w4_hbm.at[pl.ds(off * 4, C * 4)], wvm[b],
                                  wsem[b]).wait()

            @pl.loop(0, C // 16)
            def _(g):
                rows16 = idx[b][pl.ds(g * 16, 16)]
                for t in range(4):
                    vload = wvm[b][pl.ds((g * 4 + t) * 16, 16)]
                    for u in range(4):
                        j2 = t * 4 + u
                        r = _dg(rows16, jnp.full((16,), j2, jnp.int32))
                        v = _dg(vload, vpats[u])
                        plsc.addupdate_scatter(cacc, [r * 4 + lane], v,
                                               mask=mask4)

        load(0, 0)

        @pl.loop(0, (NCH - 1) // 2)
        def _(j):
            for b in (0, 1):
                k = 2 * j + b
                load(k + 1, 1 - b)
                process(k, b)

        process(NCH - 1, (NCH - 1) % 2)
        pltpu.sync_copy(cacc, cp_hbm.at[wid])

    f = pl.kernel(
        body,
        out_type=jax.ShapeDtypeStruct((NW, N * 4), jnp.float32),
        mesh=_vector_mesh(),
        compiler_params=_sc_params(),
        scratch_types=[
            pltpu.VMEM((C,), jnp.int32),
            pltpu.VMEM((C,), jnp.int32),
            pltpu.VMEM((C * 4,), jnp.float32),
            pltpu.VMEM((C * 4,), jnp.float32),
            pltpu.VMEM((N * 4,), jnp.float32),
            pltpu.SemaphoreType.DMA,
            pltpu.SemaphoreType.DMA,
        ],
    )
    return f(w4flat, row_s)


def _node_body(h_ref, m00_ref, m01_ref, m10_ref, m11_ref,
               a_ref, b_ref, bn1_ref, wn2_ref, bn2_ref, hn_ref):
    h = h_ref[...]
    mi = (m00_ref[...] + m01_ref[...]) + (m10_ref[...] + m11_ref[...])
    z = (jnp.dot(h, a_ref[...], preferred_element_type=jnp.float32)
         + jnp.dot(mi, b_ref[...], preferred_element_type=jnp.float32)
         + bn1_ref[...])
    u = z * jax.nn.sigmoid(z)
    hn_ref[...] = h + jnp.dot(u, wn2_ref[...], preferred_element_type=jnp.float32) + bn2_ref[...]


def _node(h, m00, m01, m10, m11, a, b, bn1, wn2, bn2):
    bn = 2000
    grid = (N // bn,)
    full = lambda i: (0, 0)
    blk = lambda i: (i, 0)
    return pl.pallas_call(
        _node_body,
        grid=grid,
        in_specs=[
            pl.BlockSpec((bn, D), blk),
            pl.BlockSpec((bn, D), blk),
            pl.BlockSpec((bn, D), blk),
            pl.BlockSpec((bn, D), blk),
            pl.BlockSpec((bn, D), blk),
            pl.BlockSpec((D, D), full),
            pl.BlockSpec((D, D), full),
            pl.BlockSpec((1, D), full),
            pl.BlockSpec((D, D), full),
            pl.BlockSpec((1, D), full),
        ],
        out_specs=pl.BlockSpec((bn, D), blk),
        out_shape=jax.ShapeDtypeStruct((N, D), jnp.float32),
    )(h, m00, m01, m10, m11, a, b, bn1, wn2, bn2)


def _coord_body(x4_ref, cp0_ref, cp1_ref, xn_ref):
    xn_ref[...] = (x4_ref[...] + jnp.sum(cp0_ref[...], axis=0)
                   + jnp.sum(cp1_ref[...], axis=0))


def _coord(x4flat, cp0, cp1):
    return pl.pallas_call(
        _coord_body,
        grid=(1,),
        in_specs=[
            pl.BlockSpec((N * 4,), lambda i: (0,)),
            pl.BlockSpec((NW, N * 4), lambda i: (0, 0)),
            pl.BlockSpec((NW, N * 4), lambda i: (0, 0)),
        ],
        out_specs=pl.BlockSpec((N * 4,), lambda i: (0,)),
        out_shape=jax.ShapeDtypeStruct((N * 4,), jnp.float32),
    )(x4flat, cp0, cp1)


def kernel(h, x, edge_index, edge_attr,
           W_e1, b_e1, W_e2, b_e2,
           W_n1, b_n1, W_n2, b_n2,
           W_c1, b_c1, W_c2, b_c2):
    row = edge_index[0]
    col = edge_index[1]
    x4flat = jnp.pad(x, ((0, 0), (0, 1))).reshape(N * 4)
    wab = jnp.concatenate([W_e1[:D], W_e1[D:2 * D]], axis=1)
    wea = W_e1[2 * D:2 * D + 16]
    wd = W_e1[2 * D + 16:].reshape(1, D)
    z128 = jnp.zeros((NSTRIPE, D), jnp.float32)
    seq1 = jnp.arange(NS * 2 * CS1, dtype=jnp.int32)
    seq2 = jnp.arange(NS * 2 * CS2, dtype=jnp.int32)

    p, q = _pq(h, wab)

    ew = (wea, wd, b_e1.reshape(1, D), W_e2, b_e2.reshape(1, D),
          W_c1, b_c1.reshape(1, D), W_c2.reshape(1, D), b_c2.reshape(1, 1))

    g1, dxf1 = _gather(ES1, CS1, p, q, x4flat, row[:ES1], col[:ES1], seq1)
    g2, dxf2 = _gather(ES2, CS2, p, q, x4flat, row[ES1:], col[ES1:], seq2)
    m1, w1 = _edge(ES1, g1, dxf1.reshape(ES1, 4), edge_attr[:ES1], *ew)
    m2, w2 = _edge(ES2, g2, dxf2.reshape(ES2, 4), edge_attr[ES1:], *ew)
    mp1 = _scatter(ES1, CS1, m1, row[:ES1], z128)
    mp2 = _scatter(ES2, CS2, m2, row[ES1:], z128)
    cp1 = _cscatter(ES1, CS1, w1.reshape(ES1 * 4), row[:ES1])
    cp2 = _cscatter(ES2, CS2, w2.reshape(ES2 * 4), row[ES1:])

    hn = _node(h, mp1[0, :N], mp1[1, :N], mp2[0, :N], mp2[1, :N],
               W_n1[:D], W_n1[D:], b_n1.reshape(1, D),
               W_n2, b_n2.reshape(1, D))
    xn = _coord(x4flat, cp1, cp2)
    return (hn, xn.reshape(N, 4)[:, :3])


# speedup vs baseline: 1.4764x; 1.0687x over previous
import dataclasses
import functools

import jax
import jax.numpy as jnp
from jax import lax
from jax.experimental import pallas as pl
from jax.experimental.pallas import tpu as pltpu
from jax.experimental.pallas import tpu_sc as plsc

N = 10000
E = 320000
D = 128
NC = 2
NS = 16
NW = NC * NS
NP = 10240
NSTRIPE = NP // NS
ES1 = 192000
ES2 = E - ES1
CS1 = 48
CS2 = 32


def _sc_params():
    cp = pltpu.CompilerParams()
    if "needs_layout_passes" in pltpu.CompilerParams.__dataclass_fields__:
        cp = dataclasses.replace(cp, needs_layout_passes=False)
    return cp


def _dg(v, idx16):
    return lax.gather(
        v, idx16[:, None],
        lax.GatherDimensionNumbers(offset_dims=(), collapsed_slice_dims=(0,),
                                   start_index_map=(0,)),
        (1,), mode=lax.GatherScatterMode.PROMISE_IN_BOUNDS)


_mesh_cache = []


def _vector_mesh():
    if not _mesh_cache:
        _mesh_cache.append(
            plsc.VectorSubcoreMesh(core_axis_name="c", subcore_axis_name="s"))
    return _mesh_cache[0]


def _pq_body(h_ref, wab_ref, p_ref, q_ref):
    pq = jnp.dot(h_ref[...], wab_ref[...], preferred_element_type=jnp.float32)
    p_ref[...] = pq[:, :D]
    q_ref[...] = pq[:, D:]


def _pq(h, wab):
    bn = 2000
    grid = (N // bn,)
    return pl.pallas_call(
        _pq_body,
        grid=grid,
        in_specs=[
            pl.BlockSpec((bn, D), lambda i: (i, 0)),
            pl.BlockSpec((D, 2 * D), lambda i: (0, 0)),
        ],
        out_specs=[
            pl.BlockSpec((bn, D), lambda i: (i, 0)),
            pl.BlockSpec((bn, D), lambda i: (i, 0)),
        ],
        out_shape=[
            jax.ShapeDtypeStruct((N, D), jnp.float32),
            jax.ShapeDtypeStruct((N, D), jnp.float32),
        ],
    )(h, wab)


def _gather(ES, C, p, q, x4flat, row_s, col_s, seq):
    EPS = ES // NW
    NCH = EPS // C

    def body(p_hbm, q_hbm, x4_hbm, row_hbm, col_hbm, seq_hbm,
             g_hbm, dx_hbm,
             idra, idca, sqg0, sqg1,
             pbuf0, pbuf1, qbuf0, qbuf1, x4v, dxbuf0, dxbuf1,
             g_sh, psem0, psem1, qsem0, qsem1,
             dxsem0, dxsem1, wbsem0, wbsem1):
        cid = lax.axis_index("c")
        sid = lax.axis_index("s")
        base = (sid * NC + cid) * EPS
        sqg = (sqg0, sqg1)
        pbuf = (pbuf0, pbuf1)
        qbuf = (qbuf0, qbuf1)
        dxbuf = (dxbuf0, dxbuf1)
        psem = (psem0, psem1)
        qsem = (qsem0, qsem1)
        dxsem = (dxsem0, dxsem1)
        wbsem = (wbsem0, wbsem1)
        sreg = (sid * 2 * C, sid * 2 * C + C)
        pltpu.sync_copy(seq_hbm.at[pl.ds(sreg[0], C)], sqg0)
        pltpu.sync_copy(seq_hbm.at[pl.ds(sreg[1], C)], sqg1)
        pltpu.sync_copy(x4_hbm, x4v)
        pltpu.sync_copy(row_hbm.at[pl.ds(base, EPS)], idra)
        pltpu.sync_copy(col_hbm.at[pl.ds(base, EPS)], idca)
        iota16 = lax.iota(jnp.int32, 16)
        lane = iota16 & 3
        pats = [u * 4 + (iota16 >> 2) for u in range(4)]

        def issue(k, b):
            pltpu.async_copy(p_hbm.at[idra.at[pl.ds(k * C, C)]], pbuf[b],
                             psem[b])
            pltpu.async_copy(q_hbm.at[idca.at[pl.ds(k * C, C)]], qbuf[b],
                             qsem[b])

        def process(k, b):
            off = base + k * C
            offp = off - 2 * C

            @pl.when(k >= 2)
            def _():
                pltpu.make_async_copy(
                    dxbuf[b], dx_hbm.at[pl.ds(offp * 4, C * 4)],
                    dxsem[b]).wait()
                pltpu.make_async_copy(
                    g_sh.at[pl.ds(sreg[b], C)], g_hbm.at[pl.ds(offp, C)],
                    wbsem[b]).wait()

            @pl.loop(0, C // 16)
            def _(g):
                r16 = idra[pl.ds(k * C + g * 16, 16)]
                c16 = idca[pl.ds(k * C + g * 16, 16)]
                for u in range(4):
                    pos_r = _dg(r16, pats[u]) * 4 + lane
                    pos_c = _dg(c16, pats[u]) * 4 + lane
                    xr = plsc.load_gather(x4v, [pos_r])
                    xc = plsc.load_gather(x4v, [pos_c])
                    dxbuf[b][pl.ds((g * 4 + u) * 16, 16)] = xr - xc

            pltpu.async_copy(dxbuf[b], dx_hbm.at[pl.ds(off * 4, C * 4)],
                             dxsem[b])
            pltpu.make_async_copy(p_hbm.at[idra.at[pl.ds(k * C, C)]],
                                  pbuf[b], psem[b]).wait()
            pltpu.sync_copy(pbuf[b], g_sh.at[pl.ds(sreg[b], C)])
            pltpu.make_async_copy(q_hbm.at[idca.at[pl.ds(k * C, C)]],
                                  qbuf[b], qsem[b]).wait()
            pltpu.sync_copy(qbuf[b], g_sh.at[sqg[b]], add=True)
            pltpu.async_copy(g_sh.at[pl.ds(sreg[b], C)],
                             g_hbm.at[pl.ds(off, C)], wbsem[b])

        issue(0, 0)

        @pl.loop(0, (NCH - 1) // 2)
        def _(j):
            for b in (0, 1):
                k = 2 * j + b
                issue(k + 1, 1 - b)
                process(k, b)

        process(NCH - 1, (NCH - 1) % 2)
        for kk in (NCH - 2, NCH - 1):
            bb = kk % 2
            offk = base + kk * C
            pltpu.make_async_copy(
                dxbuf[bb], dx_hbm.at[pl.ds(offk * 4, C * 4)],
                dxsem[bb]).wait()
            pltpu.make_async_copy(
                g_sh.at[pl.ds(sreg[bb], C)], g_hbm.at[pl.ds(offk, C)],
                wbsem[bb]).wait()

    f = pl.kernel(
        body,
        out_type=[
            jax.ShapeDtypeStruct((ES, D), jnp.float32),
            jax.ShapeDtypeStruct((ES * 4,), jnp.float32),
        ],
        mesh=_vector_mesh(),
        compiler_params=_sc_params(),
        scratch_types=[
            pltpu.VMEM((EPS,), jnp.int32),
            pltpu.VMEM((EPS,), jnp.int32),
            pltpu.VMEM((C,), jnp.int32),
            pltpu.VMEM((C,), jnp.int32),
            pltpu.VMEM((C, D), jnp.float32),
            pltpu.VMEM((C, D), jnp.float32),
            pltpu.VMEM((C, D), jnp.float32),
            pltpu.VMEM((C, D), jnp.float32),
            pltpu.VMEM((N * 4,), jnp.float32),
            pltpu.VMEM((C * 4,), jnp.float32),
            pltpu.VMEM((C * 4,), jnp.float32),
            pltpu.VMEM_SHARED((NS * 2 * C, D), jnp.float32),
            pltpu.SemaphoreType.DMA,
            pltpu.SemaphoreType.DMA,
            pltpu.SemaphoreType.DMA,
            pltpu.SemaphoreType.DMA,
            pltpu.SemaphoreType.DMA,
            pltpu.SemaphoreType.DMA,
            pltpu.SemaphoreType.DMA,
            pltpu.SemaphoreType.DMA,
        ],
    )
    return f(p, q, x4flat, row_s, col_s, seq)


def _edge_body(g_ref, dx_ref, ea_ref, wea_ref, wd_ref, be1_ref,
               we2_ref, be2_ref, wc1_ref, bc1_ref, wc2_ref, bc2_ref,
               m_ref, wdx_ref):
    dx = dx_ref[...]
    dist = jnp.sqrt(jnp.sum(dx * dx, axis=1, keepdims=True))
    z1 = (g_ref[...]
          + jnp.dot(ea_ref[...], wea_ref[...], preferred_element_type=jnp.float32)
          + dist * wd_ref[...]
          + be1_ref[...])
    a1 = z1 * jax.nn.sigmoid(z1)
    m = jnp.dot(a1, we2_ref[...], preferred_element_type=jnp.float32) + be2_ref[...]
    m_ref[...] = m
    z2 = jnp.dot(m, wc1_ref[...], preferred_element_type=jnp.float32) + bc1_ref[...]
    t = z2 * jax.nn.sigmoid(z2)
    cw = jnp.sum(t * wc2_ref[...], axis=1, keepdims=True) + bc2_ref[...]
    wdx_ref[...] = dx * cw


def _edge(ES, g, dx, ea, wea, wd, be1, we2, be2, wc1, bc1, wc2, bc2):
    be = 2000
    grid = (ES // be,)
    full = lambda i: (0, 0)
    return pl.pallas_call(
        _edge_body,
        grid=grid,
        in_specs=[
            pl.BlockSpec((be, D), lambda i: (i, 0)),
            pl.BlockSpec((be, 4), lambda i: (i, 0)),
            pl.BlockSpec((be, 16), lambda i: (i, 0)),
            pl.BlockSpec((16, D), full),
            pl.BlockSpec((1, D), full),
            pl.BlockSpec((1, D), full),
            pl.BlockSpec((D, D), full),
            pl.BlockSpec((1, D), full),
            pl.BlockSpec((D, D), full),
            pl.BlockSpec((1, D), full),
            pl.BlockSpec((1, D), full),
            pl.BlockSpec((1, 1), full),
        ],
        out_specs=[
            pl.BlockSpec((be, D), lambda i: (i, 0)),
            pl.BlockSpec((be, 4), lambda i: (i, 0)),
        ],
        out_shape=[
            jax.ShapeDtypeStruct((ES, D), jnp.float32),
            jax.ShapeDtypeStruct((ES, 4), jnp.float32),
        ],
    )(g, dx, ea, wea, wd, be1, we2, be2, wc1, bc1, wc2, bc2)


def _scatter(ES, C, m_ij, row_s, z128):
    EPS = ES // NW
    NCH = EPS // C

    def body(m_hbm, row_hbm, z128_hbm, mp_hbm, m_sh,
             idx0, idx1, mbuf0, mbuf1, msem0, msem1):
        cid = lax.axis_index("c")
        sid = lax.axis_index("s")
        base = (sid * NC + cid) * EPS
        stripe = sid * NSTRIPE
        idx = (idx0, idx1)
        mbuf = (mbuf0, mbuf1)
        msem = (msem0, msem1)
        pltpu.sync_copy(z128_hbm, m_sh.at[pl.ds(stripe, NSTRIPE)])
        plsc.subcore_barrier()

        def load(k, b):
            off = base + k * C
            pltpu.sync_copy(row_hbm.at[pl.ds(off, C)], idx[b])
            pltpu.async_copy(m_hbm.at[pl.ds(off, C)], mbuf[b], msem[b])

        def process(k, b):
            off = base + k * C
            pltpu.make_async_copy(m_hbm.at[pl.ds(off, C)], mbuf[b],
                                  msem[b]).wait()
            pltpu.sync_copy(mbuf[b], m_sh.at[idx[b]], add=True)

        load(0, 0)

        @pl.loop(0, (NCH - 1) // 2)
        def _(j):
            for b in (0, 1):
                k = 2 * j + b
                load(k + 1, 1 - b)
                process(k, b)

        process(NCH - 1, (NCH - 1) % 2)
        plsc.subcore_barrier()
        pltpu.sync_copy(m_sh.at[pl.ds(stripe, NSTRIPE)],
                        mp_hbm.at[cid, pl.ds(stripe, NSTRIPE)])

    f = pl.kernel(
        body,
        out_type=jax.ShapeDtypeStruct((NC, NP, D), jnp.float32),
        mesh=_vector_mesh(),
        scratch_types=[
            pltpu.VMEM_SHARED((NP, D), jnp.float32),
            pltpu.VMEM((C,), jnp.int32),
            pltpu.VMEM((C,), jnp.int32),
            pltpu.VMEM((C, D), jnp.float32),
            pltpu.VMEM((C, D), jnp.float32),
            pltpu.SemaphoreType.DMA,
            pltpu.SemaphoreType.DMA,
        ],
    )
    return f(m_ij, row_s, z128)


def _cscatter(ES, C, w4flat, row_s):
    EPS = ES // NW
    NCH = EPS // C

    def body(w4_hbm, row_hbm, cp_hbm, idxa, wvm0, wvm1, cacc,
             wsem0, wsem1):
        cid = lax.axis_index("c")
        sid = lax.axis_index("s")
        wid = sid * NC + cid
        base = wid * EPS
        pltpu.sync_copy(row_hbm.at[pl.ds(base, EPS)], idxa)
        wvm = (wvm0, wvm1)
        wsem = (wsem0, wsem1)
        zero16 = jnp.zeros((16,), jnp.float32)

        @pl.loop(0, (N * 4) // 16)
        def _(i):
            cacc[pl.ds(i * 16, 16)] = zero16

        iota16 = lax.iota(jnp.int32, 16)
        lane = iota16 & 3
        mask4 = iota16 < 4
        vpats = [u * 4 + lane for u in range(4)]

        def load(k, b):
            off = base + k * C
            pltpu.async_copy(w4_hbm.at[pl.ds(off * 4, C * 4)], wvm[b],
                             wsem[b])

        def process(k, b):
            off = base + k * C
            pltpu.make_async_copy(w4_hbm.at[pl.ds(off * 4, C * 4)], wvm[b],
                                  wsem[b]).wait()

            @pl.loop(0, C // 16)
            def _(g):
                rows16 = idxa[pl.ds(k * C + g * 16, 16)]
                for t in range(4):
                    vload = wvm[b][pl.ds((g * 4 + t) * 16, 16)]
                    for u in range(4):
                        j2 = t * 4 + u
                        r = _dg(rows16, jnp.full((16,), j2, jnp.int32))
                        v = _dg(vload, vpats[u])
                        plsc.addupdate_scatter(cacc, [r * 4 + lane], v,
                                               mask=mask4)

        load(0, 0)

        @pl.loop(0, (NCH - 1) // 2)
        def _(j):
            for b in (0, 1):
                k = 2 * j + b
                load(k + 1, 1 - b)
                process(k, b)

        process(NCH - 1, (NCH - 1) % 2)
        pltpu.sync_copy(cacc, cp_hbm.at[wid])

    f = pl.kernel(
        body,
        out_type=jax.ShapeDtypeStruct((NW, N * 4), jnp.float32),
        mesh=_vector_mesh(),
        compiler_params=_sc_params(),
        scratch_types=[
            pltpu.VMEM((EPS,), jnp.int32),
            pltpu.VMEM((C * 4,), jnp.float32),
            pltpu.VMEM((C * 4,), jnp.float32),
            pltpu.VMEM((N * 4,), jnp.float32),
            pltpu.SemaphoreType.DMA,
            pltpu.SemaphoreType.DMA,
        ],
    )
    return f(w4flat, row_s)


def _node_body(h_ref, m00_ref, m01_ref, m10_ref, m11_ref,
               a_ref, b_ref, bn1_ref, wn2_ref, bn2_ref, hn_ref):
    h = h_ref[...]
    mi = (m00_ref[...] + m01_ref[...]) + (m10_ref[...] + m11_ref[...])
    z = (jnp.dot(h, a_ref[...], preferred_element_type=jnp.float32)
         + jnp.dot(mi, b_ref[...], preferred_element_type=jnp.float32)
         + bn1_ref[...])
    u = z * jax.nn.sigmoid(z)
    hn_ref[...] = h + jnp.dot(u, wn2_ref[...], preferred_element_type=jnp.float32) + bn2_ref[...]


def _node(h, m00, m01, m10, m11, a, b, bn1, wn2, bn2):
    bn = 2000
    grid = (N // bn,)
    full = lambda i: (0, 0)
    blk = lambda i: (i, 0)
    return pl.pallas_call(
        _node_body,
        grid=grid,
        in_specs=[
            pl.BlockSpec((bn, D), blk),
            pl.BlockSpec((bn, D), blk),
            pl.BlockSpec((bn, D), blk),
            pl.BlockSpec((bn, D), blk),
            pl.BlockSpec((bn, D), blk),
            pl.BlockSpec((D, D), full),
            pl.BlockSpec((D, D), full),
            pl.BlockSpec((1, D), full),
            pl.BlockSpec((D, D), full),
            pl.BlockSpec((1, D), full),
        ],
        out_specs=pl.BlockSpec((bn, D), blk),
        out_shape=jax.ShapeDtypeStruct((N, D), jnp.float32),
    )(h, m00, m01, m10, m11, a, b, bn1, wn2, bn2)


def _coord_body(x4_ref, cp0_ref, cp1_ref, xn_ref):
    xn_ref[...] = (x4_ref[...] + jnp.sum(cp0_ref[...], axis=0)
                   + jnp.sum(cp1_ref[...], axis=0))


def _coord(x4flat, cp0, cp1):
    return pl.pallas_call(
        _coord_body,
        grid=(1,),
        in_specs=[
            pl.BlockSpec((N * 4,), lambda i: (0,)),
            pl.BlockSpec((NW, N * 4), lambda i: (0, 0)),
            pl.BlockSpec((NW, N * 4), lambda i: (0, 0)),
        ],
        out_specs=pl.BlockSpec((N * 4,), lambda i: (0,)),
        out_shape=jax.ShapeDtypeStruct((N * 4,), jnp.float32),
    )(x4flat, cp0, cp1)


def kernel(h, x, edge_index, edge_attr,
           W_e1, b_e1, W_e2, b_e2,
           W_n1, b_n1, W_n2, b_n2,
           W_c1, b_c1, W_c2, b_c2):
    row = edge_index[0]
    col = edge_index[1]
    x4flat = jnp.pad(x, ((0, 0), (0, 1))).reshape(N * 4)
    wab = jnp.concatenate([W_e1[:D], W_e1[D:2 * D]], axis=1)
    wea = W_e1[2 * D:2 * D + 16]
    wd = W_e1[2 * D + 16:].reshape(1, D)
    z128 = jnp.zeros((NSTRIPE, D), jnp.float32)
    seq1 = jnp.arange(NS * 2 * CS1, dtype=jnp.int32)
    seq2 = jnp.arange(NS * 2 * CS2, dtype=jnp.int32)

    p, q = _pq(h, wab)

    ew = (wea, wd, b_e1.reshape(1, D), W_e2, b_e2.reshape(1, D),
          W_c1, b_c1.reshape(1, D), W_c2.reshape(1, D), b_c2.reshape(1, 1))

    g1, dxf1 = _gather(ES1, CS1, p, q, x4flat, row[:ES1], col[:ES1], seq1)
    g2, dxf2 = _gather(ES2, CS2, p, q, x4flat, row[ES1:], col[ES1:], seq2)
    m1, w1 = _edge(ES1, g1, dxf1.reshape(ES1, 4), edge_attr[:ES1], *ew)
    m2, w2 = _edge(ES2, g2, dxf2.reshape(ES2, 4), edge_attr[ES1:], *ew)
    mp1 = _scatter(ES1, CS1, m1, row[:ES1], z128)
    mp2 = _scatter(ES2, CS2, m2, row[ES1:], z128)
    cp1 = _cscatter(ES1, CS1, w1.reshape(ES1 * 4), row[:ES1])
    cp2 = _cscatter(ES2, CS2, w2.reshape(ES2 * 4), row[ES1:])

    hn = _node(h, mp1[0, :N], mp1[1, :N], mp2[0, :N], mp2[1, :N],
               W_n1[:D], W_n1[D:], b_n1.reshape(1, D),
               W_n2, b_n2.reshape(1, D))
    xn = _coord(x4flat, cp1, cp2)
    return (hn, xn.reshape(N, 4)[:, :3])
